# baseline (speedup 1.0000x reference)
import functools

import jax
import jax.numpy as jnp
from jax.experimental import pallas as pl
from jax.experimental.pallas import tpu as pltpu

_BF = jnp.bfloat16


def _mega_kernel(nb, bm,
                 xp_hbm, xn_hbm, adj_hbm, diff_hbm,
                 w1_ref, w2_ref, b_ref, al_ref,
                 oa1p, oa1n, od1p, od1n, oa2p, oa2n, od2p, od2n,
                 f1a, f1d, f2a, f2d, a_buf, hp_buf, hn_buf,
                 a_sem, x_sem, o_sem):
    s = pl.program_id(0)
    total = 1 + 4 * nb
    dh = oa1p.shape[1]

    def start_a_copy(t):
        @pl.when((t >= 1) & (t <= 4 * nb))
        def _():
            pt = (t - 1) // nb
            off = ((t - 1) % nb) * bm
            slot = t % 2

            @pl.when((pt == 0) | (pt == 2))
            def _():
                pltpu.make_async_copy(adj_hbm.at[pl.ds(off, bm), :],
                                      a_buf.at[slot], a_sem.at[slot]).start()

            @pl.when((pt == 1) | (pt == 3))
            def _():
                pltpu.make_async_copy(diff_hbm.at[pl.ds(off, bm), :],
                                      a_buf.at[slot], a_sem.at[slot]).start()

    start_a_copy(s + 1)

    @pl.when(s == 0)
    def _():
        cp = pltpu.make_async_copy(xp_hbm, f2a, x_sem.at[0])
        cn = pltpu.make_async_copy(xn_hbm, f2d, x_sem.at[1])
        cp.start()
        cn.start()
        cp.wait()
        cn.wait()
        n_rows = f1a.shape[0]
        ch = n_rows // 4
        for r in range(4):
            rows = pl.ds(r * ch, ch)
            xp = f2a[rows, :]
            xn = f2d[rows, :]
            for dst, wi in ((f1a, 0), (f1d, 1)):
                w = w1_ref[wi]
                dst[rows, :] = jnp.concatenate(
                    [jnp.dot(xp, w, preferred_element_type=jnp.float32),
                     jnp.dot(xn, w, preferred_element_type=jnp.float32)],
                    axis=1).astype(_BF)

    @pl.when(s >= 1)
    def _():
        p = (s - 1) // nb
        i = (s - 1) % nb
        slot = s % 2
        oslot = s % 2

        pltpu.make_async_copy(adj_hbm.at[pl.ds(0, bm), :],
                              a_buf.at[slot], a_sem.at[slot]).wait()

        @pl.when(s >= 3)
        def _():
            pltpu.make_async_copy(hp_buf.at[oslot], oa1p.at[pl.ds(0, bm), :],
                                  o_sem.at[oslot, 0]).wait()
            pltpu.make_async_copy(hn_buf.at[oslot], oa1n.at[pl.ds(0, bm), :],
                                  o_sem.at[oslot, 1]).wait()

        b_row = b_ref[pl.ds(p, 1), :]
        al_row = al_ref[pl.ds(p, 1), :]

        def gcn_phase(f_ref, f2_ref, w2i, op, on):
            n_k = a_buf.shape[2]
            kch = n_k // 2
            h = None
            for kk in range(2):
                kds = pl.ds(kk * kch, kch)
                part = jnp.dot(a_buf[slot, :, kds].astype(_BF),
                               f_ref[kds, :],
                               preferred_element_type=jnp.float32)
                h = part if h is None else h + part
            h = h + b_row
            h = jnp.where(h >= 0, h, h * al_row)
            if f2_ref is not None:
                f2_ref[pl.ds(i * bm, bm), :] = jnp.dot(
                    h.astype(_BF), w2_ref[w2i],
                    preferred_element_type=jnp.float32).astype(_BF)
            hp_buf[oslot] = h[:, :dh]
            hn_buf[oslot] = h[:, dh:]
            pltpu.make_async_copy(hp_buf.at[oslot],
                                  op.at[pl.ds(i * bm, bm), :],
                                  o_sem.at[oslot, 0]).start()
            pltpu.make_async_copy(hn_buf.at[oslot],
                                  on.at[pl.ds(i * bm, bm), :],
                                  o_sem.at[oslot, 1]).start()

        for ph, (f_ref, f2_ref, w2i, op, on) in enumerate((
                (f1a, f2a, 0, oa1p, oa1n),
                (f1d, f2d, 1, od1p, od1n),
                (f2a, None, None, oa2p, oa2n),
                (f2d, None, None, od2p, od2n))):
            @pl.when(p == ph)
            def _(f_ref=f_ref, f2_ref=f2_ref, w2i=w2i, op=op, on=on):
                gcn_phase(f_ref, f2_ref, w2i, op, on)

        @pl.when(s == total - 1)
        def _():
            other = (oslot + 1) % 2
            for sl in (oslot, other):
                pltpu.make_async_copy(hp_buf.at[sl], oa1p.at[pl.ds(0, bm), :],
                                      o_sem.at[sl, 0]).wait()
                pltpu.make_async_copy(hn_buf.at[sl], oa1n.at[pl.ds(0, bm), :],
                                      o_sem.at[sl, 1]).wait()


def _pair_row(b, a):
    b128 = jnp.concatenate([b, b])
    a128 = jnp.broadcast_to(a.reshape(1), (2 * b.shape[0],))
    return b128, a128


def _blockdiag2(wt):
    k, h = wt.shape
    z = jnp.zeros((k, h), wt.dtype)
    return jnp.concatenate(
        [jnp.concatenate([wt, z], axis=1),
         jnp.concatenate([z, wt], axis=1)], axis=0)


def kernel(seq_pos, seq_neg, adj, diff, sparse, msk, samp_bias1, samp_bias2,
           W_adj1, b_adj1, a_adj1, W_diff1, b_diff1, a_diff1,
           W_adj2, b_adj2, a_adj2, W_diff2, b_diff2, a_diff2):
    n = seq_pos.shape[1]
    din = seq_pos.shape[2]
    dh = W_adj1.shape[0]
    c = 2 * dh
    bm = 400 if n % 400 == 0 else 16
    nb = n // bm

    a2 = adj.reshape(n, n)
    d2 = diff.reshape(n, n)
    xp = seq_pos.reshape(n, din).astype(_BF)
    xn = seq_neg.reshape(n, din).astype(_BF)

    w1 = jnp.stack([W_adj1.T.astype(_BF), W_diff1.T.astype(_BF)])
    w2 = jnp.stack([_blockdiag2(W_adj2.T.astype(_BF)),
                    _blockdiag2(W_diff2.T.astype(_BF))])
    rows = [_pair_row(b_adj1, a_adj1), _pair_row(b_diff1, a_diff1),
            _pair_row(b_adj2, a_adj2), _pair_row(b_diff2, a_diff2)]
    b_all = jnp.stack([r[0] for r in rows])
    al_all = jnp.stack([r[1] for r in rows])

    hbm = pl.BlockSpec(memory_space=pltpu.MemorySpace.HBM)
    out = jax.ShapeDtypeStruct((n, dh), jnp.float32)
    outs = pl.pallas_call(
        functools.partial(_mega_kernel, nb, bm),
        grid=(1 + 4 * nb,),
        in_specs=[
            hbm, hbm, hbm, hbm,
            pl.BlockSpec((2, din, dh), lambda s: (0, 0, 0)),
            pl.BlockSpec((2, c, c), lambda s: (0, 0, 0)),
            pl.BlockSpec((4, c), lambda s: (0, 0)),
            pl.BlockSpec((4, c), lambda s: (0, 0)),
        ],
        out_specs=[hbm] * 8,
        out_shape=[out] * 8,
        scratch_shapes=[
            pltpu.VMEM((n, c), _BF),
            pltpu.VMEM((n, c), _BF),
            pltpu.VMEM((n, c), _BF),
            pltpu.VMEM((n, c), _BF),
            pltpu.VMEM((2, bm, n), jnp.float32),
            pltpu.VMEM((2, bm, dh), jnp.float32),
            pltpu.VMEM((2, bm, dh), jnp.float32),
            pltpu.SemaphoreType.DMA((2,)),
            pltpu.SemaphoreType.DMA((2,)),
            pltpu.SemaphoreType.DMA((2, 2)),
        ],
        compiler_params=pltpu.CompilerParams(
            vmem_limit_bytes=64 * 1024 * 1024),
    )(xp, xn, a2, d2, w1, w2, b_all, al_all)

    oa1p, oa1n, od1p, od1n, oa2p, oa2n, od2p, od2n = outs

    def lift(h):
        return h[None]

    return (lift(oa1p), lift(od1p), lift(oa2p), lift(od2p),
            lift(oa1n), lift(od1n), lift(oa2n), lift(od2n))


# speedup vs baseline: 1.0018x; 1.0018x over previous
import functools

import jax
import jax.numpy as jnp
from jax.experimental import pallas as pl
from jax.experimental.pallas import tpu as pltpu

_BF = jnp.bfloat16


def _mega_kernel(nb, bm,
                 xp_hbm, xn_hbm, adj_hbm, diff_hbm,
                 w1_ref, w2_ref, b_ref, al_ref,
                 oa1p, oa1n, od1p, od1n, oa2p, oa2n, od2p, od2n,
                 f1a, f1d, f2a, f2d, a_buf, hp_buf, hn_buf,
                 a_sem, x_sem, o_sem):
    s = pl.program_id(0)
    total = 1 + 4 * nb
    dh = oa1p.shape[1]

    def start_a_copy(t):
        @pl.when((t >= 1) & (t <= 4 * nb))
        def _():
            pt = (t - 1) // nb
            off = ((t - 1) % nb) * bm
            slot = t % 2

            @pl.when((pt == 0) | (pt == 2))
            def _():
                pltpu.make_async_copy(adj_hbm.at[pl.ds(off, bm), :],
                                      a_buf.at[slot], a_sem.at[slot]).start()

            @pl.when((pt == 1) | (pt == 3))
            def _():
                pltpu.make_async_copy(diff_hbm.at[pl.ds(off, bm), :],
                                      a_buf.at[slot], a_sem.at[slot]).start()

    start_a_copy(s + 1)

    @pl.when(s == 0)
    def _():
        cp = pltpu.make_async_copy(xp_hbm, f2a, x_sem.at[0])
        cn = pltpu.make_async_copy(xn_hbm, f2d, x_sem.at[1])
        cp.start()
        cn.start()
        cp.wait()
        cn.wait()
        n_rows = f1a.shape[0]
        ch = n_rows // 4
        for r in range(4):
            rows = pl.ds(r * ch, ch)
            xp = f2a[rows, :]
            xn = f2d[rows, :]
            for dst, wi in ((f1a, 0), (f1d, 1)):
                w = w1_ref[wi]
                dst[rows, :] = jnp.concatenate(
                    [jnp.dot(xp, w, preferred_element_type=jnp.float32),
                     jnp.dot(xn, w, preferred_element_type=jnp.float32)],
                    axis=1).astype(_BF)

    @pl.when(s >= 1)
    def _():
        p = (s - 1) // nb
        i = (s - 1) % nb
        slot = s % 2
        oslot = s % 2

        pltpu.make_async_copy(adj_hbm.at[pl.ds(0, bm), :],
                              a_buf.at[slot], a_sem.at[slot]).wait()

        @pl.when(s >= 3)
        def _():
            pltpu.make_async_copy(hp_buf.at[oslot], oa1p.at[pl.ds(0, bm), :],
                                  o_sem.at[oslot, 0]).wait()
            pltpu.make_async_copy(hn_buf.at[oslot], oa1n.at[pl.ds(0, bm), :],
                                  o_sem.at[oslot, 1]).wait()

        b_row = b_ref[pl.ds(p, 1), :]
        al_row = al_ref[pl.ds(p, 1), :]

        def gcn_phase(f_ref, f2_ref, w2i, op, on):
            n_k = a_buf.shape[2]
            kch = n_k // 4
            h = None
            for kk in range(4):
                kds = pl.ds(kk * kch, kch)
                part = jnp.dot(a_buf[slot, :, kds].astype(_BF),
                               f_ref[kds, :],
                               preferred_element_type=jnp.float32)
                h = part if h is None else h + part
            h = h + b_row
            h = jnp.where(h >= 0, h, h * al_row)
            if f2_ref is not None:
                f2_ref[pl.ds(i * bm, bm), :] = jnp.dot(
                    h.astype(_BF), w2_ref[w2i],
                    preferred_element_type=jnp.float32).astype(_BF)
            hp_buf[oslot] = h[:, :dh]
            hn_buf[oslot] = h[:, dh:]
            pltpu.make_async_copy(hp_buf.at[oslot],
                                  op.at[pl.ds(i * bm, bm), :],
                                  o_sem.at[oslot, 0]).start()
            pltpu.make_async_copy(hn_buf.at[oslot],
                                  on.at[pl.ds(i * bm, bm), :],
                                  o_sem.at[oslot, 1]).start()

        for ph, (f_ref, f2_ref, w2i, op, on) in enumerate((
                (f1a, f2a, 0, oa1p, oa1n),
                (f1d, f2d, 1, od1p, od1n),
                (f2a, None, None, oa2p, oa2n),
                (f2d, None, None, od2p, od2n))):
            @pl.when(p == ph)
            def _(f_ref=f_ref, f2_ref=f2_ref, w2i=w2i, op=op, on=on):
                gcn_phase(f_ref, f2_ref, w2i, op, on)

        @pl.when(s == total - 1)
        def _():
            other = (oslot + 1) % 2
            for sl in (oslot, other):
                pltpu.make_async_copy(hp_buf.at[sl], oa1p.at[pl.ds(0, bm), :],
                                      o_sem.at[sl, 0]).wait()
                pltpu.make_async_copy(hn_buf.at[sl], oa1n.at[pl.ds(0, bm), :],
                                      o_sem.at[sl, 1]).wait()


def _pair_row(b, a):
    b128 = jnp.concatenate([b, b])
    a128 = jnp.broadcast_to(a.reshape(1), (2 * b.shape[0],))
    return b128, a128


def _blockdiag2(wt):
    k, h = wt.shape
    z = jnp.zeros((k, h), wt.dtype)
    return jnp.concatenate(
        [jnp.concatenate([wt, z], axis=1),
         jnp.concatenate([z, wt], axis=1)], axis=0)


def kernel(seq_pos, seq_neg, adj, diff, sparse, msk, samp_bias1, samp_bias2,
           W_adj1, b_adj1, a_adj1, W_diff1, b_diff1, a_diff1,
           W_adj2, b_adj2, a_adj2, W_diff2, b_diff2, a_diff2):
    n = seq_pos.shape[1]
    din = seq_pos.shape[2]
    dh = W_adj1.shape[0]
    c = 2 * dh
    bm = 400 if n % 400 == 0 else 16
    nb = n // bm

    a2 = adj.reshape(n, n)
    d2 = diff.reshape(n, n)
    xp = seq_pos.reshape(n, din).astype(_BF)
    xn = seq_neg.reshape(n, din).astype(_BF)

    w1 = jnp.stack([W_adj1.T.astype(_BF), W_diff1.T.astype(_BF)])
    w2 = jnp.stack([_blockdiag2(W_adj2.T.astype(_BF)),
                    _blockdiag2(W_diff2.T.astype(_BF))])
    rows = [_pair_row(b_adj1, a_adj1), _pair_row(b_diff1, a_diff1),
            _pair_row(b_adj2, a_adj2), _pair_row(b_diff2, a_diff2)]
    b_all = jnp.stack([r[0] for r in rows])
    al_all = jnp.stack([r[1] for r in rows])

    hbm = pl.BlockSpec(memory_space=pltpu.MemorySpace.HBM)
    out = jax.ShapeDtypeStruct((n, dh), jnp.float32)
    outs = pl.pallas_call(
        functools.partial(_mega_kernel, nb, bm),
        grid=(1 + 4 * nb,),
        in_specs=[
            hbm, hbm, hbm, hbm,
            pl.BlockSpec((2, din, dh), lambda s: (0, 0, 0)),
            pl.BlockSpec((2, c, c), lambda s: (0, 0, 0)),
            pl.BlockSpec((4, c), lambda s: (0, 0)),
            pl.BlockSpec((4, c), lambda s: (0, 0)),
        ],
        out_specs=[hbm] * 8,
        out_shape=[out] * 8,
        scratch_shapes=[
            pltpu.VMEM((n, c), _BF),
            pltpu.VMEM((n, c), _BF),
            pltpu.VMEM((n, c), _BF),
            pltpu.VMEM((n, c), _BF),
            pltpu.VMEM((2, bm, n), jnp.float32),
            pltpu.VMEM((2, bm, dh), jnp.float32),
            pltpu.VMEM((2, bm, dh), jnp.float32),
            pltpu.SemaphoreType.DMA((2,)),
            pltpu.SemaphoreType.DMA((2,)),
            pltpu.SemaphoreType.DMA((2, 2)),
        ],
        compiler_params=pltpu.CompilerParams(
            vmem_limit_bytes=64 * 1024 * 1024),
    )(xp, xn, a2, d2, w1, w2, b_all, al_all)

    oa1p, oa1n, od1p, od1n, oa2p, oa2n, od2p, od2n = outs

    def lift(h):
        return h[None]

    return (lift(oa1p), lift(od1p), lift(oa2p), lift(od2p),
            lift(oa1n), lift(od1n), lift(oa2n), lift(od2n))
